# Initial kernel scaffold; baseline (speedup 1.0000x reference)
import jax
import jax.numpy as jnp
from jax.experimental import pallas as pl


def kernel(position, positions, neighbor_map, embeddings):
    raise NotImplementedError("write your pallas kernel here")


# speedup vs baseline: 11.5383x; 11.5383x over previous
import jax
import jax.numpy as jnp
from jax import lax
from jax.experimental import pallas as pl
from jax.experimental.pallas import tpu as pltpu
from jax.experimental.pallas import tpu_sc as plsc

Q = 65536
N_POS = 4096
EMB = 64
K_NN = 4
GRID = 256

NC = 2
NS = 16
NW = NC * NS
QW = Q // NW
NBQ = 128
NBLK = QW // NBQ
NG = NBQ // 16


def _rsqrt(s):
    i = lax.bitcast_convert_type(s, jnp.int32)
    i = jnp.int32(0x5F3759DF) - (i >> 1)
    y = lax.bitcast_convert_type(i, jnp.float32)
    for _ in range(3):
        y = y * (1.5 - 0.5 * s * y * y)
    return y


def _full16(v):
    return jnp.full((16,), v, dtype=jnp.int32)


def _sc_body(px_hbm, py_hbm, pxa_hbm, pya_hbm, nmf_hbm, emb_hbm, out_hbm,
             pxq_v, pyq_v, pxa_v, pya_v, cellk_v, nmr_v, coef_v,
             rows_v, out_v, sem):
    c = lax.axis_index("c")
    s = lax.axis_index("s")
    wid = s * NC + c
    base = wid * QW

    pltpu.sync_copy(px_hbm.at[pl.ds(base, QW)], pxq_v)
    pltpu.sync_copy(py_hbm.at[pl.ds(base, QW)], pyq_v)
    pltpu.sync_copy(pxa_hbm, pxa_v)
    pltpu.sync_copy(pya_hbm, pya_v)

    def block(b, carry):
        qb = b * NBQ

        def grp1(j, carry):
            qx = pxq_v[pl.ds(qb + j * 16, 16)]
            qy = pyq_v[pl.ds(qb + j * 16, 16)]
            ix = qx.astype(jnp.int32)
            iy = qy.astype(jnp.int32)
            cell4 = (ix * GRID + iy) * K_NN
            for k in range(K_NN):
                cellk_v[pl.ds(k * NBQ + j * 16, 16)] = cell4 + k
            return carry

        lax.fori_loop(0, NG, grp1, 0)

        nm_descs = [
            pltpu.async_copy(nmf_hbm.at[cellk_v.at[pl.ds(k * NBQ, NBQ)]],
                             nmr_v.at[pl.ds(k * NBQ, NBQ)], sem)
            for k in range(K_NN)
        ]
        for d in nm_descs:
            d.wait()

        def grp2(j, carry):
            qx = pxq_v[pl.ds(qb + j * 16, 16)]
            qy = pyq_v[pl.ds(qb + j * 16, 16)]
            ixf = qx.astype(jnp.int32).astype(jnp.float32)
            iyf = qy.astype(jnp.int32).astype(jnp.float32)
            ws = []
            for k in range(K_NN):
                nk = nmr_v[pl.ds(k * NBQ + j * 16, 16)]
                ax = plsc.load_gather(pxa_v, [nk])
                ay = plsc.load_gather(pya_v, [nk])
                dx = ax - ixf
                dy = ay - iyf
                s2 = dx * dx + dy * dy
                w = jnp.where(s2 == 0.0, jnp.float32(1e8), _rsqrt(s2))
                ws.append(w)
            wsum = (ws[0] + ws[1]) + (ws[2] + ws[3])
            scale = jnp.where(wsum > 1000.0, 1.0 / wsum, jnp.float32(0.0))
            for k in range(K_NN):
                coef_v[pl.ds(k * NBQ + j * 16, 16)] = ws[k] * scale
            return carry

        lax.fori_loop(0, NG, grp2, 0)

        row_descs = [
            pltpu.async_copy(emb_hbm.at[nmr_v.at[pl.ds(k * NBQ, NBQ)]],
                             rows_v.at[k], sem)
            for k in range(K_NN)
        ]
        for d in row_descs:
            d.wait()

        def comb(q, carry):
            cs = [plsc.load_gather(coef_v, [_full16(k * NBQ) + q])
                  for k in range(K_NN)]
            for e in range(EMB // 16):
                acc = cs[0] * rows_v[0, q, pl.ds(e * 16, 16)]
                for k in range(1, K_NN):
                    acc = acc + cs[k] * rows_v[k, q, pl.ds(e * 16, 16)]
                out_v[q, pl.ds(e * 16, 16)] = acc
            return carry

        lax.fori_loop(0, NBQ, comb, 0)

        pltpu.sync_copy(out_v, out_hbm.at[pl.ds(base + qb, NBQ)])
        return carry

    lax.fori_loop(0, NBLK, block, 0)


@jax.jit
def _latent_map_sc(px, py, pxa, pya, nmf, emb):
    mesh = plsc.VectorSubcoreMesh(
        core_axis_name="c", subcore_axis_name="s",
        num_cores=NC, num_subcores=NS)
    return pl.kernel(
        _sc_body,
        out_type=jax.ShapeDtypeStruct((Q, EMB), jnp.float32),
        mesh=mesh,
        compiler_params=pltpu.CompilerParams(
            needs_layout_passes=False, use_tc_tiling_on_sc=False),
        scratch_types=[
            pltpu.VMEM((QW,), jnp.float32),
            pltpu.VMEM((QW,), jnp.float32),
            pltpu.VMEM((N_POS,), jnp.float32),
            pltpu.VMEM((N_POS,), jnp.float32),
            pltpu.VMEM((K_NN * NBQ,), jnp.int32),
            pltpu.VMEM((K_NN * NBQ,), jnp.int32),
            pltpu.VMEM((K_NN * NBQ,), jnp.float32),
            pltpu.VMEM((K_NN, NBQ, EMB), jnp.float32),
            pltpu.VMEM((NBQ, EMB), jnp.float32),
            pltpu.SemaphoreType.DMA,
        ],
    )(px, py, pxa, pya, nmf, emb)


def kernel(position, positions, neighbor_map, embeddings):
    px = position[:, 0]
    py = position[:, 1]
    pxa = positions[:, 0]
    pya = positions[:, 1]
    nmf = neighbor_map.reshape(Q * K_NN)
    return _latent_map_sc(px, py, pxa, pya, nmf, embeddings)


# speedup vs baseline: 11.5650x; 1.0023x over previous
import jax
import jax.numpy as jnp
from jax import lax
from jax.experimental import pallas as pl
from jax.experimental.pallas import tpu as pltpu
from jax.experimental.pallas import tpu_sc as plsc

Q = 65536
N_POS = 4096
EMB = 64
K_NN = 4
GRID = 256

NC = 2
NS = 16
NW = NC * NS
QW = Q // NW
NBQ = 128
NBLK = QW // NBQ
NG = NBQ // 16


def _rsqrt(s):
    i = lax.bitcast_convert_type(s, jnp.int32)
    i = jnp.int32(0x5F3759DF) - (i >> 1)
    y = lax.bitcast_convert_type(i, jnp.float32)
    for _ in range(3):
        y = y * (1.5 - 0.5 * s * y * y)
    return y


def _full16(v):
    return jnp.full((16,), v, dtype=jnp.int32)


def _sc_body(px_hbm, py_hbm, pxa_hbm, pya_hbm, nmf_hbm, emb_hbm, out_hbm,
             pxq_v, pyq_v, pxa_v, pya_v, cellk_v, nmr_v, coef_v,
             rows_v, out_v, sem):
    c = lax.axis_index("c")
    s = lax.axis_index("s")
    wid = s * NC + c
    base = wid * QW

    pltpu.sync_copy(px_hbm.at[pl.ds(base, QW)], pxq_v)
    pltpu.sync_copy(py_hbm.at[pl.ds(base, QW)], pyq_v)
    pltpu.sync_copy(pxa_hbm, pxa_v)
    pltpu.sync_copy(pya_hbm, pya_v)

    def block(b, carry):
        qb = b * NBQ

        def grp1(j, carry):
            qx = pxq_v[pl.ds(qb + j * 16, 16)]
            qy = pyq_v[pl.ds(qb + j * 16, 16)]
            ix = qx.astype(jnp.int32)
            iy = qy.astype(jnp.int32)
            cell4 = (ix * GRID + iy) * K_NN
            for k in range(K_NN):
                cellk_v[pl.ds(k * NBQ + j * 16, 16)] = cell4 + k
            return carry

        lax.fori_loop(0, NG, grp1, 0)

        nm_descs = [
            pltpu.async_copy(nmf_hbm.at[cellk_v.at[pl.ds(k * NBQ, NBQ)]],
                             nmr_v.at[pl.ds(k * NBQ, NBQ)], sem)
            for k in range(K_NN)
        ]
        for d in nm_descs:
            d.wait()

        def grp2(j, carry):
            qx = pxq_v[pl.ds(qb + j * 16, 16)]
            qy = pyq_v[pl.ds(qb + j * 16, 16)]
            ixf = qx.astype(jnp.int32).astype(jnp.float32)
            iyf = qy.astype(jnp.int32).astype(jnp.float32)
            ws = []
            for k in range(K_NN):
                nk = nmr_v[pl.ds(k * NBQ + j * 16, 16)]
                ax = plsc.load_gather(pxa_v, [nk])
                ay = plsc.load_gather(pya_v, [nk])
                dx = ax - ixf
                dy = ay - iyf
                s2 = dx * dx + dy * dy
                w = jnp.where(s2 == 0.0, jnp.float32(1e8), _rsqrt(s2))
                ws.append(w)
            wsum = (ws[0] + ws[1]) + (ws[2] + ws[3])
            scale = jnp.where(wsum > 1000.0, 1.0 / wsum, jnp.float32(0.0))
            for k in range(K_NN):
                coef_v[pl.ds(k * NBQ + j * 16, 16)] = ws[k] * scale
            return carry

        lax.fori_loop(0, NG, grp2, 0)

        row_descs = [
            pltpu.async_copy(emb_hbm.at[nmr_v.at[pl.ds(k * NBQ, NBQ)]],
                             rows_v.at[k], sem)
            for k in range(K_NN)
        ]
        for d in row_descs:
            d.wait()

        def comb(q, carry):
            cs = [plsc.load_gather(coef_v, [_full16(k * NBQ) + q])
                  for k in range(K_NN)]
            for e in range(EMB // 16):
                acc = cs[0] * rows_v[0, q, pl.ds(e * 16, 16)]
                for k in range(1, K_NN):
                    acc = acc + cs[k] * rows_v[k, q, pl.ds(e * 16, 16)]
                out_v[pl.ds(q * EMB + e * 16, 16)] = acc
            return carry

        lax.fori_loop(0, NBQ, comb, 0)

        pltpu.sync_copy(out_v, out_hbm.at[pl.ds((base + qb) * EMB, NBQ * EMB)])
        return carry

    lax.fori_loop(0, NBLK, block, 0)


@jax.jit
def _latent_map_sc(px, py, pxa, pya, nmf, emb):
    mesh = plsc.VectorSubcoreMesh(
        core_axis_name="c", subcore_axis_name="s",
        num_cores=NC, num_subcores=NS)
    return pl.kernel(
        _sc_body,
        out_type=jax.ShapeDtypeStruct((Q * EMB,), jnp.float32),
        mesh=mesh,
        compiler_params=pltpu.CompilerParams(
            needs_layout_passes=False, use_tc_tiling_on_sc=False),
        scratch_types=[
            pltpu.VMEM((QW,), jnp.float32),
            pltpu.VMEM((QW,), jnp.float32),
            pltpu.VMEM((N_POS,), jnp.float32),
            pltpu.VMEM((N_POS,), jnp.float32),
            pltpu.VMEM((K_NN * NBQ,), jnp.int32),
            pltpu.VMEM((K_NN * NBQ,), jnp.int32),
            pltpu.VMEM((K_NN * NBQ,), jnp.float32),
            pltpu.VMEM((K_NN, NBQ, EMB), jnp.float32),
            pltpu.VMEM((NBQ * EMB,), jnp.float32),
            pltpu.SemaphoreType.DMA,
        ],
    )(px, py, pxa, pya, nmf, emb)


def kernel(position, positions, neighbor_map, embeddings):
    px = position[:, 0]
    py = position[:, 1]
    pxa = positions[:, 0]
    pya = positions[:, 1]
    nmf = neighbor_map.reshape(Q * K_NN)
    out = _latent_map_sc(px, py, pxa, pya, nmf, embeddings)
    return out.reshape(Q, EMB)
